# baseline (speedup 1.0000x reference)
import functools

import numpy as np
import jax
import jax.numpy as jnp
from jax import lax
from jax.experimental import pallas as pl
from jax.experimental.pallas import tpu as pltpu
from jax.experimental.pallas import tpu_sc as plsc

_B = 16384
_D = 26
_K = 1000
_RW = 32
_GB = 16
_GROUP = _GB * _D
_OUTW = _GB * _D * _D
_NW = 32
_BPW = _B // _NW
_NG = _BPW // _GB
_GCH = 104

_n = np.arange(_GROUP, dtype=np.int32)
_JPAT = np.asarray((_n % _D) * _K, dtype=np.int32)
_KP = 1008


def _sc_gather(dwt, mean2d, lv2d, labels_flat, jpat):
    mesh = plsc.VectorSubcoreMesh(core_axis_name="c", subcore_axis_name="s")

    @functools.partial(
        pl.kernel,
        out_type=[
            jax.ShapeDtypeStruct((_B * _D * _D,), jnp.float32),
            jax.ShapeDtypeStruct((_B * _D,), jnp.float32),
            jax.ShapeDtypeStruct((_B * _D,), jnp.float32),
            jax.ShapeDtypeStruct((2 * _D * _K, _RW), jnp.float32),
        ],
        mesh=mesh,
        compiler_params=pltpu.CompilerParams(
            needs_layout_passes=False, use_tc_tiling_on_sc=False),
        scratch_types=(
            [pltpu.VMEM((_GROUP,), jnp.int32)] * 2
            + [pltpu.VMEM((_GROUP,), jnp.int32)] * 2
            + [pltpu.VMEM((_GROUP, _RW), jnp.float32)] * 2
            + [pltpu.VMEM((_OUTW,), jnp.float32)] * 2
            + [pltpu.VMEM((_GROUP,), jnp.float32)] * 2
            + [pltpu.VMEM((_GROUP,), jnp.float32)] * 2
            + [pltpu.VMEM((_GROUP,), jnp.int32)]
            + [pltpu.VMEM((_D, 1, _KP), jnp.float32)]
            + [pltpu.VMEM((1, _KP), jnp.float32)] * 2
            + [pltpu.VMEM((_K, _RW), jnp.float32)]
            + [pltpu.SemaphoreType.DMA] * 6
        ),
    )
    def k(dwt_hbm, mean_hbm2, lv_hbm2, lab_hbm, jpat_hbm,
          dw_hbm, mean_hbm, lv_hbm, tbl_hbm,
          lbuf0, lbuf1, idx0, idx1, rbuf0, rbuf1, obuf0, obuf1,
          mbuf0, mbuf1, vbuf0, vbuf1, jpat_v, dbuf, mrow, lrow, tbuf,
          sl0, sl1, sg0, sg1, so0, so1):
        core = lax.axis_index("c")
        sub = lax.axis_index("s")
        lanes16 = lax.iota(jnp.int32, 16)

        for rep in range(2):
            j = sub + 16 * rep

            @pl.when(j < _D)
            def _():
                pltpu.sync_copy(dwt_hbm.at[:, pl.ds(j, 1), :],
                                dbuf.at[:, :, pl.ds(0, _K)])
                pltpu.sync_copy(mean_hbm2.at[pl.ds(j, 1), :],
                                mrow.at[:, pl.ds(0, _K)])
                pltpu.sync_copy(lv_hbm2.at[pl.ds(j, 1), :],
                                lrow.at[:, pl.ds(0, _K)])
                def kv_body(kv, c):
                    rows = lanes16 + kv * 16
                    mask = rows < _K
                    for i in range(_D):
                        v = dbuf[i, 0, pl.ds(kv * 16, 16)]
                        plsc.store_scatter(
                            tbuf, [rows, jnp.full((16,), i, jnp.int32)], v,
                            mask=mask)
                    vm = mrow[0, pl.ds(kv * 16, 16)]
                    plsc.store_scatter(
                        tbuf, [rows, jnp.full((16,), _D, jnp.int32)], vm,
                        mask=mask)
                    vl = lrow[0, pl.ds(kv * 16, 16)]
                    plsc.store_scatter(
                        tbuf, [rows, jnp.full((16,), _D + 1, jnp.int32)], vl,
                        mask=mask)
                    return c
                lax.fori_loop(0, _KP // 16, kv_body, 0)
                pltpu.sync_copy(
                    tbuf, tbl_hbm.at[pl.ds((core * _D + j) * _K, _K)])
        plsc.subcore_barrier()

        lbuf = (lbuf0, lbuf1)
        idxb = (idx0, idx1)
        rbuf = (rbuf0, rbuf1)
        obuf = (obuf0, obuf1)
        mbuf = (mbuf0, mbuf1)
        vbuf = (vbuf0, vbuf1)
        slab = (sl0, sl1)
        sgat = (sg0, sg1)
        sout = (so0, so1)

        wid = lax.axis_index("s") * 2 + lax.axis_index("c")
        b0 = wid * _BPW
        pltpu.sync_copy(jpat_hbm, jpat_v)
        lanes = lax.iota(jnp.int32, 16)
        cvec = lanes * _D
        c2vec = cvec + 16 * _D
        m10 = lanes < (_D - 16)
        col_mean = jnp.full((16,), _D, jnp.int32)
        col_lv = jnp.full((16,), _D + 1, jnp.int32)

        def lab_slice(g):
            return lab_hbm.at[pl.ds((b0 + g * _GB) * _D, _GROUP)]

        def fire_labels(g, p):
            return pltpu.async_copy(lab_slice(g), lbuf[p], slab[p])

        def wait_labels(g, p):
            pltpu.make_async_copy(lab_slice(g), lbuf[p], slab[p]).wait()

        cbias = core * (_D * _K)

        def compute_idx(p):
            def body(v, c):
                s = pl.ds(v * 16, 16)
                idxb[p][s] = lbuf[p][s] + jpat_v[s] + cbias
                return c
            lax.fori_loop(0, _GROUP // 16, body, 0)

        def fire_gathers(p):
            for c in range(_GROUP // _GCH):
                s = pl.ds(c * _GCH, _GCH)
                pltpu.async_copy(tbl_hbm.at[idxb[p].at[s]],
                                 rbuf[p].at[s], sgat[p])

        def wait_gathers(p):
            for c in range(_GROUP // _GCH):
                s = pl.ds(c * _GCH, _GCH)
                pltpu.make_async_copy(tbl_hbm.at[idxb[p].at[s]],
                                      rbuf[p].at[s], sgat[p]).wait()

        def transpose(p):
            def bl_body(bl, c):
                obase = bl * (_D * _D)
                rbase = bl * _D
                for j in range(_D):
                    v1 = rbuf[p][rbase + j, pl.ds(0, 16)]
                    v2 = rbuf[p][rbase + j, pl.ds(16, 16)]
                    plsc.store_scatter(obuf[p], [obase + j + cvec], v1)
                    plsc.store_scatter(obuf[p], [obase + j + c2vec], v2,
                                       mask=m10)
                return c
            lax.fori_loop(0, _GB, bl_body, 0)

            def mv_body(v, c):
                s = pl.ds(v * 16, 16)
                rows = lanes + v * 16
                mbuf[p][s] = plsc.load_gather(rbuf[p], [rows, col_mean])
                vbuf[p][s] = plsc.load_gather(rbuf[p], [rows, col_lv])
                return c
            lax.fori_loop(0, _GROUP // 16, mv_body, 0)

        def out_slices(g):
            base_b = b0 + g * _GB
            return (dw_hbm.at[pl.ds(base_b * _D * _D, _OUTW)],
                    mean_hbm.at[pl.ds(base_b * _D, _GROUP)],
                    lv_hbm.at[pl.ds(base_b * _D, _GROUP)])

        def fire_out(g, p):
            dws, ms, vs = out_slices(g)
            pltpu.async_copy(obuf[p], dws, sout[p])
            pltpu.async_copy(mbuf[p], ms, sout[p])
            pltpu.async_copy(vbuf[p], vs, sout[p])

        def wait_out(g, p):
            dws, ms, vs = out_slices(g)
            pltpu.make_async_copy(obuf[p], dws, sout[p]).wait()
            pltpu.make_async_copy(mbuf[p], ms, sout[p]).wait()
            pltpu.make_async_copy(vbuf[p], vs, sout[p]).wait()

        fire_labels(0, 0)
        fire_labels(1, 1)
        wait_labels(0, 0)
        compute_idx(0)
        fire_gathers(0)

        def halfstep(g, p, q):
            @pl.when(g + 1 <= _NG - 1)
            def _():
                wait_labels(g + 1, q)
                compute_idx(q)
                fire_gathers(q)
            wait_gathers(p)

            @pl.when(g >= 2)
            def _():
                wait_out(g - 2, p)
            transpose(p)
            fire_out(g, p)

            @pl.when(g + 2 <= _NG - 1)
            def _():
                fire_labels(g + 2, p)

        def step(gg, c):
            halfstep(2 * gg, 0, 1)
            halfstep(2 * gg + 1, 1, 0)
            return c
        lax.fori_loop(0, _NG // 2, step, 0)
        wait_out(_NG - 2, (_NG - 2) % 2)
        wait_out(_NG - 1, (_NG - 1) % 2)

    return k(dwt, mean2d, lv2d, labels_flat, jpat)


def kernel(labels, mean, log_var, domain_weights):
    labels = labels.astype(jnp.int32)
    dwf, mf, vf, _unused_tbl = _sc_gather(
        domain_weights, mean, log_var, labels.reshape(-1),
        jnp.asarray(_JPAT))
    return (mf.reshape(_B, _D), vf.reshape(_B, _D),
            dwf.reshape(_B, _D, _D))


# speedup vs baseline: 1.9330x; 1.9330x over previous
import functools

import jax
import jax.numpy as jnp
from jax import lax
from jax.experimental import pallas as pl
from jax.experimental.pallas import tpu as pltpu
from jax.experimental.pallas import tpu_sc as plsc

_B = 16384
_D = 26
_K = 1000
_NU = 2 * _D
_HB = _B // 2
_CH = 1024
_NCH = _HB // _CH
_NW = 32


def _sc_gather(dwt2d, mean_flat, lv_flat, labels_t):
    mesh = plsc.VectorSubcoreMesh(core_axis_name="c", subcore_axis_name="s")

    @functools.partial(
        pl.kernel,
        out_type=[
            jax.ShapeDtypeStruct((_D, _D, _B), jnp.float32),
            jax.ShapeDtypeStruct((_D, _B), jnp.float32),
            jax.ShapeDtypeStruct((_D, _B), jnp.float32),
        ],
        mesh=mesh,
        compiler_params=pltpu.CompilerParams(
            needs_layout_passes=False, use_tc_tiling_on_sc=False),
        scratch_types=(
            [pltpu.VMEM((_D, _K), jnp.float32)]
            + [pltpu.VMEM((_K,), jnp.float32)] * 2
            + [pltpu.VMEM((_HB,), jnp.int32)]
            + [pltpu.VMEM((32,), jnp.int32)]
            + [pltpu.VMEM((_D, 1, _CH), jnp.float32)] * 2
            + [pltpu.VMEM((1, _CH), jnp.float32)] * 2
            + [pltpu.VMEM((1, _CH), jnp.float32)] * 2
            + [pltpu.SemaphoreType.DMA] * 3
        ),
    )
    def k(dwt_hbm, mean_hbm, lv_hbm, labt_hbm,
          dw_hbm, mt_hbm, lt_hbm,
          rows, mrow, lrow, lbuf, ridx,
          ob0, ob1, om0, om1, ol0, ol1,
          sgat, so0, so1):
        obdw = (ob0, ob1)
        obm = (om0, om1)
        obl = (ol0, ol1)
        sout = (so0, so1)

        wid = lax.axis_index("s") * 2 + lax.axis_index("c")
        lanes = lax.iota(jnp.int32, 16)
        splat_i = [jnp.full((16,), i, jnp.int32) for i in range(_D)]

        u_start = (13 * wid) // 8
        u_end = (13 * (wid + 1)) // 8

        def out_slices(j, half, c, s):
            b0 = half * _HB + c * _CH
            return (dw_hbm.at[:, pl.ds(j, 1), pl.ds(b0, _CH)],
                    mt_hbm.at[pl.ds(j, 1), pl.ds(b0, _CH)],
                    lt_hbm.at[pl.ds(j, 1), pl.ds(b0, _CH)])

        def fire_out(j, half, c, s):
            dws, ms, ls = out_slices(j, half, c, s)
            pltpu.async_copy(obdw[s], dws, sout[s])
            pltpu.async_copy(obm[s], ms, sout[s])
            pltpu.async_copy(obl[s], ls, sout[s])

        def wait_out(j, half, c, s):
            dws, ms, ls = out_slices(j, half, c, s)
            pltpu.make_async_copy(obdw[s], dws, sout[s]).wait()
            pltpu.make_async_copy(obm[s], ms, sout[s]).wait()
            pltpu.make_async_copy(obl[s], ls, sout[s]).wait()

        def chunk(j, half, c, s, first_round):
            @pl.when(jnp.logical_not(first_round))
            def _():
                wait_out(j, half, c, s)

            def v_body(v, carry):
                idxv = lbuf[pl.ds(c * _CH + v * 16, 16)]
                for i in range(_D):
                    val = plsc.load_gather(rows, [splat_i[i], idxv])
                    obdw[s][i, 0, pl.ds(v * 16, 16)] = val
                obm[s][0, pl.ds(v * 16, 16)] = plsc.load_gather(mrow, [idxv])
                obl[s][0, pl.ds(v * 16, 16)] = plsc.load_gather(lrow, [idxv])
                return carry
            lax.fori_loop(0, _CH // 16, v_body, 0)
            fire_out(j, half, c, s)

        def unit(u, carry):
            j = u // 2
            half = u - 2 * (u // 2)
            ridx[pl.ds(0, 16)] = lanes * _D + j
            ridx[pl.ds(16, 16)] = (lanes + 16) * _D + j
            pltpu.async_copy(dwt_hbm.at[ridx.at[pl.ds(0, _D)]], rows, sgat)
            pltpu.sync_copy(mean_hbm.at[pl.ds(j * _K, _K)], mrow)
            pltpu.sync_copy(lv_hbm.at[pl.ds(j * _K, _K)], lrow)
            pltpu.sync_copy(
                labt_hbm.at[pl.ds(j * _B + half * _HB, _HB)], lbuf)
            pltpu.make_async_copy(
                dwt_hbm.at[ridx.at[pl.ds(0, _D)]], rows, sgat).wait()

            first = u == u_start
            for cc in range(_NCH // 2):
                chunk(j, half, 2 * cc, 0,
                      jnp.logical_and(first, cc == 0))
                chunk(j, half, 2 * cc + 1, 1,
                      jnp.logical_and(first, cc == 0))
            return carry
        lax.fori_loop(u_start, u_end, unit, 0)

        @pl.when(u_end > u_start)
        def _():
            u_last = u_end - 1
            j = u_last // 2
            half = u_last - 2 * (u_last // 2)
            wait_out(j, half, _NCH - 2, 0)
            wait_out(j, half, _NCH - 1, 1)

    return k(dwt2d, mean_flat, lv_flat, labels_t)


def kernel(labels, mean, log_var, domain_weights):
    labels = labels.astype(jnp.int32)
    labels_t = jnp.transpose(labels).reshape(-1)
    dwp, mt, lt = _sc_gather(
        domain_weights.reshape(_D * _D, _K),
        mean.reshape(-1), log_var.reshape(-1), labels_t)
    means = jnp.transpose(mt)
    log_vars = jnp.transpose(lt)
    dw = jnp.transpose(dwp, (2, 0, 1))
    return (means, log_vars, dw)


# speedup vs baseline: 3.5274x; 1.8249x over previous
import functools

import jax
import jax.numpy as jnp
from jax import lax
from jax.experimental import pallas as pl
from jax.experimental.pallas import tpu as pltpu
from jax.experimental.pallas import tpu_sc as plsc

_B = 16384
_D = 26
_K = 1000
_NU = 2 * _D
_HB = _B // 2
_CH = 1024
_NCH = _HB // _CH
_NW = 32


def _sc_gather(dwt2d, mean_flat, lv_flat, labels_t):
    mesh = plsc.VectorSubcoreMesh(core_axis_name="c", subcore_axis_name="s")

    @functools.partial(
        pl.kernel,
        out_type=[
            jax.ShapeDtypeStruct((_D, _D, _B), jnp.float32),
            jax.ShapeDtypeStruct((_D, _B), jnp.float32),
            jax.ShapeDtypeStruct((_D, _B), jnp.float32),
        ],
        mesh=mesh,
        compiler_params=pltpu.CompilerParams(
            needs_layout_passes=False, use_tc_tiling_on_sc=False),
        scratch_types=(
            [pltpu.VMEM((_D, _K), jnp.float32)]
            + [pltpu.VMEM((_K,), jnp.float32)] * 2
            + [pltpu.VMEM((_HB,), jnp.int32)]
            + [pltpu.VMEM((32,), jnp.int32)]
            + [pltpu.VMEM((_D, 1, _CH), jnp.float32)] * 2
            + [pltpu.VMEM((1, _CH), jnp.float32)] * 2
            + [pltpu.VMEM((1, _CH), jnp.float32)] * 2
            + [pltpu.SemaphoreType.DMA] * 3
        ),
    )
    def k(dwt_hbm, mean_hbm, lv_hbm, labt_hbm,
          dw_hbm, mt_hbm, lt_hbm,
          rows, mrow, lrow, lbuf, ridx,
          ob0, ob1, om0, om1, ol0, ol1,
          sgat, so0, so1):
        obdw = (ob0, ob1)
        obm = (om0, om1)
        obl = (ol0, ol1)
        sout = (so0, so1)

        wid = lax.axis_index("s") * 2 + lax.axis_index("c")
        lanes = lax.iota(jnp.int32, 16)
        splat_i = [jnp.full((16,), i, jnp.int32) for i in range(_D)]

        u_start = (13 * wid) // 8
        u_end = (13 * (wid + 1)) // 8

        def out_slices(j, half, c, s):
            b0 = half * _HB + c * _CH
            return (dw_hbm.at[:, pl.ds(j, 1), pl.ds(b0, _CH)],
                    mt_hbm.at[pl.ds(j, 1), pl.ds(b0, _CH)],
                    lt_hbm.at[pl.ds(j, 1), pl.ds(b0, _CH)])

        def fire_out(j, half, c, s):
            dws, ms, ls = out_slices(j, half, c, s)
            pltpu.async_copy(obdw[s], dws, sout[s])
            pltpu.async_copy(obm[s], ms, sout[s])
            pltpu.async_copy(obl[s], ls, sout[s])

        def wait_out(j, half, c, s):
            dws, ms, ls = out_slices(j, half, c, s)
            pltpu.make_async_copy(obdw[s], dws, sout[s]).wait()
            pltpu.make_async_copy(obm[s], ms, sout[s]).wait()
            pltpu.make_async_copy(obl[s], ls, sout[s]).wait()

        def chunk(j, half, c, s, first_round):
            @pl.when(jnp.logical_not(first_round))
            def _():
                wait_out(j, half, c, s)

            @plsc.parallel_loop(0, _CH // 16, unroll=2)
            def v_body(v):
                idxv = lbuf[pl.ds(c * _CH + v * 16, 16)]
                for i in range(_D):
                    val = plsc.load_gather(rows, [splat_i[i], idxv])
                    obdw[s][i, 0, pl.ds(v * 16, 16)] = val
                obm[s][0, pl.ds(v * 16, 16)] = plsc.load_gather(mrow, [idxv])
                obl[s][0, pl.ds(v * 16, 16)] = plsc.load_gather(lrow, [idxv])
            fire_out(j, half, c, s)

        def unit(u, carry):
            j = u // 2
            half = u - 2 * (u // 2)
            ridx[pl.ds(0, 16)] = lanes * _D + j
            ridx[pl.ds(16, 16)] = (lanes + 16) * _D + j
            pltpu.async_copy(dwt_hbm.at[ridx.at[pl.ds(0, _D)]], rows, sgat)
            pltpu.sync_copy(mean_hbm.at[pl.ds(j * _K, _K)], mrow)
            pltpu.sync_copy(lv_hbm.at[pl.ds(j * _K, _K)], lrow)
            pltpu.sync_copy(
                labt_hbm.at[pl.ds(j * _B + half * _HB, _HB)], lbuf)
            pltpu.make_async_copy(
                dwt_hbm.at[ridx.at[pl.ds(0, _D)]], rows, sgat).wait()

            first = u == u_start
            for cc in range(_NCH // 2):
                chunk(j, half, 2 * cc, 0,
                      jnp.logical_and(first, cc == 0))
                chunk(j, half, 2 * cc + 1, 1,
                      jnp.logical_and(first, cc == 0))
            return carry
        lax.fori_loop(u_start, u_end, unit, 0)

        @pl.when(u_end > u_start)
        def _():
            u_last = u_end - 1
            j = u_last // 2
            half = u_last - 2 * (u_last // 2)
            wait_out(j, half, _NCH - 2, 0)
            wait_out(j, half, _NCH - 1, 1)

    return k(dwt2d, mean_flat, lv_flat, labels_t)


def kernel(labels, mean, log_var, domain_weights):
    labels = labels.astype(jnp.int32)
    labels_t = jnp.transpose(labels).reshape(-1)
    dwp, mt, lt = _sc_gather(
        domain_weights.reshape(_D * _D, _K),
        mean.reshape(-1), log_var.reshape(-1), labels_t)
    means = jnp.transpose(mt)
    log_vars = jnp.transpose(lt)
    dw = jnp.transpose(dwp, (2, 0, 1))
    return (means, log_vars, dw)
